# baseline (speedup 1.0000x reference)
import functools

import jax
import jax.numpy as jnp
from jax import lax
from jax.experimental import pallas as pl
from jax.experimental.pallas import tpu as pltpu
from jax.experimental.pallas import tpu_sc as plsc

F32 = jnp.float32
BF16 = jnp.bfloat16
NC, NS = 2, 16
NW = NC * NS
CH = 80


BCH = 16


@functools.lru_cache(maxsize=None)
def _make_agg(N: int, NCHW: int, D: int, count: bool):
    NBLK = NCHW // BCH
    RPT = N // NS
    NRING = 12 if count else 16
    GLAG = 8 if count else 10
    mesh = plsc.VectorSubcoreMesh(core_axis_name="c", subcore_axis_name="s")

    out_type = [jax.ShapeDtypeStruct((NC, N, D), BF16)]
    scratch = [
        pltpu.VMEM((3, BCH, CH), jnp.int32),
        pltpu.VMEM((3, BCH, CH), jnp.int32),
        pltpu.VMEM((NRING, CH, D), BF16),
        pltpu.VMEM_SHARED((N, D), BF16),
        pltpu.SemaphoreType.DMA,
        pltpu.SemaphoreType.DMA,
    ]
    if count:
        out_type.append(jax.ShapeDtypeStruct((NC, N, 16), F32))
        scratch += [
            pltpu.VMEM((CH, 16), F32),
            pltpu.VMEM_SHARED((N, 16), F32),
            pltpu.SemaphoreType.DMA,
        ]

    @functools.partial(
        pl.kernel,
        out_type=out_type,
        mesh=mesh,
        scratch_types=scratch,
        compiler_params=pltpu.CompilerParams(use_tc_tiling_on_sc=False),
    )
    def agg(p_hbm, src_hbm, dst_hbm, zero_hbm, *rest):
        if count:
            (ones_hbm, czero_hbm, out_hbm, cout_hbm,
             sidx, didx, rows, acc, gsem, ssem, onev, cacc, csem) = rest
        else:
            out_hbm, sidx, didx, rows, acc, gsem, ssem = rest
        c = lax.axis_index("c")
        s = lax.axis_index("s")
        wid = s * NC + c

        def stage_idx(blk, slot):
            pltpu.sync_copy(src_hbm.at[wid, pl.ds(blk * BCH, BCH)],
                            sidx.at[slot])
            pltpu.sync_copy(dst_hbm.at[wid, pl.ds(blk * BCH, BCH)],
                            didx.at[slot])

        pltpu.sync_copy(zero_hbm.at[pl.ds(s * RPT, RPT)],
                        acc.at[pl.ds(s * RPT, RPT)])
        if count:
            pltpu.sync_copy(czero_hbm.at[pl.ds(s * RPT, RPT)],
                            cacc.at[pl.ds(s * RPT, RPT)])
            pltpu.sync_copy(ones_hbm, onev)
        stage_idx(0, 0)
        stage_idx(1, 1)
        plsc.subcore_barrier()

        def idx_of(ch):
            return lax.rem(lax.div(ch, BCH), 3), lax.rem(ch, BCH)

        def gather(ch, b):
            slot, r = idx_of(ch)
            pltpu.async_copy(p_hbm.at[sidx.at[slot, r]], rows.at[b], gsem)

        def scatter(ch, b):
            slot, r = idx_of(ch)
            pltpu.async_copy(rows.at[b], acc.at[didx.at[slot, r]], ssem,
                             add=True)
            if count:
                pltpu.async_copy(onev, cacc.at[didx.at[slot, r]], csem,
                                 add=True)

        def wait_g():
            pltpu.make_async_copy(zero_hbm.at[pl.ds(0, CH)], rows.at[0],
                                  gsem).wait()

        def wait_s():
            pltpu.make_async_copy(zero_hbm.at[pl.ds(0, CH)], rows.at[0],
                                  ssem).wait()

        for k in range(GLAG):
            gather(k, k)

        def body(st, carry):
            blk = lax.div(st, BCH)

            @pl.when(jnp.logical_and(lax.rem(st, BCH) == 0,
                                     blk < NBLK - 1))
            def _():
                stage_idx(blk + 1, lax.rem(blk + 1, 3))

            @pl.when(st >= NRING)
            def _():
                wait_s()

            gather(st, lax.rem(st, NRING))
            wait_g()
            scatter(st - GLAG, lax.rem(st - GLAG, NRING))
            return carry

        lax.fori_loop(GLAG, NCHW, body, 0)
        for j in range(NCHW - GLAG, NCHW):
            wait_g()
            scatter(j, j % NRING)
        for _ in range(NRING):
            wait_s()
        if count:
            def cdrain(ch, carry):
                pltpu.make_async_copy(ones_hbm, onev, csem).wait()
                return carry

            lax.fori_loop(0, NCHW, cdrain, 0)

        plsc.subcore_barrier()
        pltpu.sync_copy(acc.at[pl.ds(s * RPT, RPT)],
                        out_hbm.at[c, pl.ds(s * RPT, RPT)])
        if count:
            pltpu.sync_copy(cacc.at[pl.ds(s * RPT, RPT)],
                            cout_hbm.at[c, pl.ds(s * RPT, RPT)])

    return agg


def _k0_body(x_ref, wl_ref, wr_ref, b_ref, p_ref, s_ref):
    x = x_ref[...]
    p_ref[...] = jnp.dot(x, wl_ref[...], preferred_element_type=F32).astype(BF16)
    s_ref[...] = jnp.dot(x, wr_ref[...], preferred_element_type=F32) + b_ref[...]


def _k1_body(a_ref, cnt_ref, s1_ref, wl_ref, wr_ref, b_ref,
             p_ref, s_ref, rcp_ref):
    a = a_ref[0].astype(F32) + a_ref[1].astype(F32)
    cnt = cnt_ref[0, :, :1] + cnt_ref[1, :, :1]
    rcp = 1.0 / jnp.maximum(cnt, 1.0)
    h = jnp.maximum(a * rcp + s1_ref[...], 0.0)
    p_ref[...] = jnp.dot(h, wl_ref[...], preferred_element_type=F32).astype(BF16)
    s_ref[...] = jnp.dot(h, wr_ref[...], preferred_element_type=F32) + b_ref[...]
    rcp_ref[...] = jnp.broadcast_to(rcp, (h.shape[0], 8))


def _k2_body(a_ref, s2_ref, rcp_ref, wl_ref, wr_ref, b_ref,
             p_ref, s_ref):
    a = a_ref[0].astype(F32) + a_ref[1].astype(F32)
    h = jnp.maximum(a * rcp_ref[:, :1] + s2_ref[...], 0.0)
    p_ref[...] = jnp.dot(h, wl_ref[...], preferred_element_type=F32).astype(BF16)
    s_ref[...] = jnp.dot(h, wr_ref[...], preferred_element_type=F32) + b_ref[...]


def _k3_body(a_ref, s3_ref, rcp_ref, out_ref):
    z = (a_ref[0].astype(F32) + a_ref[1].astype(F32)) * rcp_ref[:, :1] + s3_ref[...]
    m = jnp.max(z, axis=1, keepdims=True)
    lse = jnp.log(jnp.sum(jnp.exp(z - m), axis=1, keepdims=True)) + m
    out_ref[...] = z - lse


def _row_spec(rb, d):
    return pl.BlockSpec((rb, d), lambda i: (i, 0))


def _acc_spec(rb, d):
    return pl.BlockSpec((NC, rb, d), lambda i: (0, i, 0))


def _full_spec(*shape):
    return pl.BlockSpec(shape, lambda i: tuple(0 for _ in shape))


def kernel(x, edge_index, Wl1, Wr1, b1, Wl2, Wr2, b2, Wl3, Wr3, b3):
    N0, DIN = x.shape
    E = edge_index.shape[1]
    DH = Wl1.shape[1]
    DOUT = Wl3.shape[1]
    N = ((N0 + NS * 8 - 1) // (NS * 8)) * (NS * 8)
    ROWB = N // 8
    grid = (N // ROWB,)
    if N != N0:
        x = jnp.concatenate([x, jnp.zeros((N - N0, DIN), x.dtype)], axis=0)

    EW = E // NW
    NCHW = ((EW + BCH * CH - 1) // (BCH * CH)) * BCH
    src = edge_index[0].reshape(NW, EW)
    dst = edge_index[1].reshape(NW, EW)
    if NCHW * CH != EW:
        npadn = N - N0
        pad_vals = N0 + (jnp.arange(NW * (NCHW * CH - EW), dtype=jnp.int32)
                         % npadn).reshape(NW, NCHW * CH - EW)
        src = jnp.concatenate([src, pad_vals], axis=1)
        dst = jnp.concatenate([dst, pad_vals], axis=1)
    src = src.reshape(NW, NCHW, CH)
    dst = dst.reshape(NW, NCHW, CH)

    k0 = pl.pallas_call(
        _k0_body,
        grid=grid,
        in_specs=[_row_spec(ROWB, DIN), _full_spec(DIN, DH), _full_spec(DIN, DH),
                  _full_spec(1, DH)],
        out_specs=[_row_spec(ROWB, DH), _row_spec(ROWB, DH)],
        out_shape=[jax.ShapeDtypeStruct((N, DH), BF16),
                   jax.ShapeDtypeStruct((N, DH), F32)],
    )
    P1, S1 = k0(x, Wl1, Wr1, b1.reshape(1, DH))

    zeros_f = jnp.zeros((N, DH), BF16)
    A1, CNT = _make_agg(N, NCHW, DH, True)(
        P1, src, dst, zeros_f,
        jnp.ones((CH, 16), F32), jnp.zeros((N, 16), F32))
    agg_f = _make_agg(N, NCHW, DH, False)

    k1 = pl.pallas_call(
        _k1_body,
        grid=grid,
        in_specs=[_acc_spec(ROWB, DH), _acc_spec(ROWB, 16),
                  _row_spec(ROWB, DH), _full_spec(DH, DH),
                  _full_spec(DH, DH), _full_spec(1, DH)],
        out_specs=[_row_spec(ROWB, DH), _row_spec(ROWB, DH), _row_spec(ROWB, 8)],
        out_shape=[jax.ShapeDtypeStruct((N, DH), BF16),
                   jax.ShapeDtypeStruct((N, DH), F32),
                   jax.ShapeDtypeStruct((N, 8), F32)],
    )
    P2, S2, RCP = k1(A1, CNT, S1, Wl2, Wr2, b2.reshape(1, DH))

    A2, = agg_f(P2, src, dst, zeros_f)

    k2 = pl.pallas_call(
        _k2_body,
        grid=grid,
        in_specs=[_acc_spec(ROWB, DH),
                  _row_spec(ROWB, DH), _row_spec(ROWB, 8),
                  _full_spec(DH, DOUT), _full_spec(DH, DOUT),
                  _full_spec(1, DOUT)],
        out_specs=[_row_spec(ROWB, DOUT), _row_spec(ROWB, DOUT)],
        out_shape=[jax.ShapeDtypeStruct((N, DOUT), BF16),
                   jax.ShapeDtypeStruct((N, DOUT), F32)],
    )
    P3, S3 = k2(A2, S2, RCP, Wl3, Wr3, b3.reshape(1, DOUT))

    A3, = _make_agg(N, NCHW, DOUT, False)(P3, src, dst,
                                          jnp.zeros((N, DOUT), BF16))

    k3 = pl.pallas_call(
        _k3_body,
        grid=grid,
        in_specs=[_acc_spec(ROWB, DOUT), _row_spec(ROWB, DOUT), _row_spec(ROWB, 8)],
        out_specs=_row_spec(ROWB, DOUT),
        out_shape=jax.ShapeDtypeStruct((N, DOUT), F32),
    )
    return k3(A3, S3, RCP)[:N0]


# speedup vs baseline: 1.0146x; 1.0146x over previous
import functools

import jax
import jax.numpy as jnp
from jax import lax
from jax.experimental import pallas as pl
from jax.experimental.pallas import tpu as pltpu
from jax.experimental.pallas import tpu_sc as plsc

F32 = jnp.float32
BF16 = jnp.bfloat16
NC, NS = 2, 16
NW = NC * NS
CH = 80


BCH = 16


@functools.lru_cache(maxsize=None)
def _make_count(N: int, NCHW: int):
    RPT = N // NS
    mesh = plsc.VectorSubcoreMesh(core_axis_name="c", subcore_axis_name="s")

    @functools.partial(
        pl.kernel,
        out_type=jax.ShapeDtypeStruct((NC, N, 16), F32),
        mesh=mesh,
        scratch_types=[
            pltpu.VMEM((NCHW, CH), jnp.int32),
            pltpu.VMEM((CH, 16), F32),
            pltpu.VMEM_SHARED((N, 16), F32),
            pltpu.SemaphoreType.DMA,
        ],
        compiler_params=pltpu.CompilerParams(use_tc_tiling_on_sc=False),
    )
    def cnt_kernel(dst_hbm, zero_hbm, ones_hbm, out_hbm, didx, onev, acc, ssem):
        c = lax.axis_index("c")
        s = lax.axis_index("s")
        wid = s * NC + c
        pltpu.sync_copy(zero_hbm.at[pl.ds(s * RPT, RPT)],
                        acc.at[pl.ds(s * RPT, RPT)])
        pltpu.sync_copy(ones_hbm, onev)
        pltpu.sync_copy(dst_hbm.at[wid], didx)
        plsc.subcore_barrier()

        def fire(ch, carry):
            pltpu.async_copy(onev, acc.at[didx.at[ch]], ssem, add=True)
            return carry

        lax.fori_loop(0, NCHW, fire, 0)

        def drain(ch, carry):
            pltpu.make_async_copy(ones_hbm, onev, ssem).wait()
            return carry

        lax.fori_loop(0, NCHW, drain, 0)
        plsc.subcore_barrier()
        pltpu.sync_copy(acc.at[pl.ds(s * RPT, RPT)],
                        out_hbm.at[c, pl.ds(s * RPT, RPT)])

    return cnt_kernel


@functools.lru_cache(maxsize=None)
def _make_agg(N: int, NCHW: int, D: int, count: bool):
    NBLK = NCHW // BCH
    RPT = N // NS
    NRING = 12 if count else 16
    GLAG = 8 if count else 10
    mesh = plsc.VectorSubcoreMesh(core_axis_name="c", subcore_axis_name="s")

    out_type = [jax.ShapeDtypeStruct((NC, N, D), BF16)]
    scratch = [
        pltpu.VMEM((3, BCH, CH), jnp.int32),
        pltpu.VMEM((3, BCH, CH), jnp.int32),
        pltpu.VMEM((NRING, CH, D), BF16),
        pltpu.VMEM_SHARED((N, D), BF16),
        pltpu.SemaphoreType.DMA,
        pltpu.SemaphoreType.DMA,
    ]
    if count:
        out_type.append(jax.ShapeDtypeStruct((NC, N, 16), F32))
        scratch += [
            pltpu.VMEM((CH, 16), F32),
            pltpu.VMEM_SHARED((N, 16), F32),
            pltpu.SemaphoreType.DMA,
        ]

    @functools.partial(
        pl.kernel,
        out_type=out_type,
        mesh=mesh,
        scratch_types=scratch,
        compiler_params=pltpu.CompilerParams(use_tc_tiling_on_sc=False),
    )
    def agg(p_hbm, src_hbm, dst_hbm, zero_hbm, *rest):
        if count:
            (ones_hbm, czero_hbm, out_hbm, cout_hbm,
             sidx, didx, rows, acc, gsem, ssem, onev, cacc, csem) = rest
        else:
            out_hbm, sidx, didx, rows, acc, gsem, ssem = rest
        c = lax.axis_index("c")
        s = lax.axis_index("s")
        wid = s * NC + c

        def stage_idx(blk, slot):
            pltpu.sync_copy(src_hbm.at[wid, pl.ds(blk * BCH, BCH)],
                            sidx.at[slot])
            pltpu.sync_copy(dst_hbm.at[wid, pl.ds(blk * BCH, BCH)],
                            didx.at[slot])

        pltpu.sync_copy(zero_hbm.at[pl.ds(s * RPT, RPT)],
                        acc.at[pl.ds(s * RPT, RPT)])
        if count:
            pltpu.sync_copy(czero_hbm.at[pl.ds(s * RPT, RPT)],
                            cacc.at[pl.ds(s * RPT, RPT)])
            pltpu.sync_copy(ones_hbm, onev)
        stage_idx(0, 0)
        stage_idx(1, 1)
        plsc.subcore_barrier()

        def idx_of(ch):
            return lax.rem(lax.div(ch, BCH), 3), lax.rem(ch, BCH)

        def gather(ch, b):
            slot, r = idx_of(ch)
            pltpu.async_copy(p_hbm.at[sidx.at[slot, r]], rows.at[b], gsem)

        def scatter(ch, b):
            slot, r = idx_of(ch)
            pltpu.async_copy(rows.at[b], acc.at[didx.at[slot, r]], ssem,
                             add=True)
            if count:
                pltpu.async_copy(onev, cacc.at[didx.at[slot, r]], csem,
                                 add=True)

        def wait_g():
            pltpu.make_async_copy(zero_hbm.at[pl.ds(0, CH)], rows.at[0],
                                  gsem).wait()

        def wait_s():
            pltpu.make_async_copy(zero_hbm.at[pl.ds(0, CH)], rows.at[0],
                                  ssem).wait()

        for k in range(GLAG):
            gather(k, k)

        def body(st, carry):
            blk = lax.div(st, BCH)

            @pl.when(jnp.logical_and(lax.rem(st, BCH) == 0,
                                     blk < NBLK - 1))
            def _():
                stage_idx(blk + 1, lax.rem(blk + 1, 3))

            @pl.when(st >= NRING)
            def _():
                wait_s()

            gather(st, lax.rem(st, NRING))
            wait_g()
            scatter(st - GLAG, lax.rem(st - GLAG, NRING))
            return carry

        lax.fori_loop(GLAG, NCHW, body, 0)
        for j in range(NCHW - GLAG, NCHW):
            wait_g()
            scatter(j, j % NRING)
        for _ in range(NRING):
            wait_s()
        if count:
            def cdrain(ch, carry):
                pltpu.make_async_copy(ones_hbm, onev, csem).wait()
                return carry

            lax.fori_loop(0, NCHW, cdrain, 0)

        plsc.subcore_barrier()
        pltpu.sync_copy(acc.at[pl.ds(s * RPT, RPT)],
                        out_hbm.at[c, pl.ds(s * RPT, RPT)])
        if count:
            pltpu.sync_copy(cacc.at[pl.ds(s * RPT, RPT)],
                            cout_hbm.at[c, pl.ds(s * RPT, RPT)])

    return agg


def _k0_body(x_ref, wl_ref, wr_ref, b_ref, p_ref, s_ref):
    x = x_ref[...]
    p_ref[...] = jnp.dot(x, wl_ref[...], preferred_element_type=F32).astype(BF16)
    s_ref[...] = jnp.dot(x, wr_ref[...], preferred_element_type=F32) + b_ref[...]


def _k1_body(a_ref, cnt_ref, s1_ref, wl_ref, wr_ref, b_ref,
             p_ref, s_ref, rcp_ref):
    a = a_ref[0].astype(F32) + a_ref[1].astype(F32)
    cnt = cnt_ref[0, :, :1] + cnt_ref[1, :, :1]
    rcp = 1.0 / jnp.maximum(cnt, 1.0)
    h = jnp.maximum(a * rcp + s1_ref[...], 0.0)
    p_ref[...] = jnp.dot(h, wl_ref[...], preferred_element_type=F32).astype(BF16)
    s_ref[...] = jnp.dot(h, wr_ref[...], preferred_element_type=F32) + b_ref[...]
    rcp_ref[...] = jnp.broadcast_to(rcp, (h.shape[0], 8))


def _k2_body(a_ref, s2_ref, rcp_ref, wl_ref, wr_ref, b_ref,
             p_ref, s_ref):
    a = a_ref[0].astype(F32) + a_ref[1].astype(F32)
    h = jnp.maximum(a * rcp_ref[:, :1] + s2_ref[...], 0.0)
    p_ref[...] = jnp.dot(h, wl_ref[...], preferred_element_type=F32).astype(BF16)
    s_ref[...] = jnp.dot(h, wr_ref[...], preferred_element_type=F32) + b_ref[...]


def _k3_body(a_ref, s3_ref, rcp_ref, out_ref):
    z = (a_ref[0].astype(F32) + a_ref[1].astype(F32)) * rcp_ref[:, :1] + s3_ref[...]
    m = jnp.max(z, axis=1, keepdims=True)
    lse = jnp.log(jnp.sum(jnp.exp(z - m), axis=1, keepdims=True)) + m
    out_ref[...] = z - lse


def _row_spec(rb, d):
    return pl.BlockSpec((rb, d), lambda i: (i, 0))


def _acc_spec(rb, d):
    return pl.BlockSpec((NC, rb, d), lambda i: (0, i, 0))


def _full_spec(*shape):
    return pl.BlockSpec(shape, lambda i: tuple(0 for _ in shape))


def kernel(x, edge_index, Wl1, Wr1, b1, Wl2, Wr2, b2, Wl3, Wr3, b3):
    N0, DIN = x.shape
    E = edge_index.shape[1]
    DH = Wl1.shape[1]
    DOUT = Wl3.shape[1]
    N = ((N0 + NS * 8 - 1) // (NS * 8)) * (NS * 8)
    ROWB = N // 8
    grid = (N // ROWB,)
    if N != N0:
        x = jnp.concatenate([x, jnp.zeros((N - N0, DIN), x.dtype)], axis=0)

    EW = E // NW
    NCHW = ((EW + BCH * CH - 1) // (BCH * CH)) * BCH
    src = edge_index[0].reshape(NW, EW)
    dst = edge_index[1].reshape(NW, EW)
    if NCHW * CH != EW:
        npadn = N - N0
        pad_vals = N0 + (jnp.arange(NW * (NCHW * CH - EW), dtype=jnp.int32)
                         % npadn).reshape(NW, NCHW * CH - EW)
        src = jnp.concatenate([src, pad_vals], axis=1)
        dst = jnp.concatenate([dst, pad_vals], axis=1)
    src = src.reshape(NW, NCHW, CH)
    dst = dst.reshape(NW, NCHW, CH)

    k0 = pl.pallas_call(
        _k0_body,
        grid=grid,
        in_specs=[_row_spec(ROWB, DIN), _full_spec(DIN, DH), _full_spec(DIN, DH),
                  _full_spec(1, DH)],
        out_specs=[_row_spec(ROWB, DH), _row_spec(ROWB, DH)],
        out_shape=[jax.ShapeDtypeStruct((N, DH), BF16),
                   jax.ShapeDtypeStruct((N, DH), F32)],
    )
    P1, S1 = k0(x, Wl1, Wr1, b1.reshape(1, DH))

    CNT = _make_count(N, NCHW)(dst, jnp.zeros((N, 16), F32),
                               jnp.ones((CH, 16), F32))
    zeros_f = jnp.zeros((N, DH), BF16)
    agg_f = _make_agg(N, NCHW, DH, False)
    A1, = agg_f(P1, src, dst, zeros_f)

    k1 = pl.pallas_call(
        _k1_body,
        grid=grid,
        in_specs=[_acc_spec(ROWB, DH), _acc_spec(ROWB, 16),
                  _row_spec(ROWB, DH), _full_spec(DH, DH),
                  _full_spec(DH, DH), _full_spec(1, DH)],
        out_specs=[_row_spec(ROWB, DH), _row_spec(ROWB, DH), _row_spec(ROWB, 8)],
        out_shape=[jax.ShapeDtypeStruct((N, DH), BF16),
                   jax.ShapeDtypeStruct((N, DH), F32),
                   jax.ShapeDtypeStruct((N, 8), F32)],
    )
    P2, S2, RCP = k1(A1, CNT, S1, Wl2, Wr2, b2.reshape(1, DH))

    A2, = agg_f(P2, src, dst, zeros_f)

    k2 = pl.pallas_call(
        _k2_body,
        grid=grid,
        in_specs=[_acc_spec(ROWB, DH),
                  _row_spec(ROWB, DH), _row_spec(ROWB, 8),
                  _full_spec(DH, DOUT), _full_spec(DH, DOUT),
                  _full_spec(1, DOUT)],
        out_specs=[_row_spec(ROWB, DOUT), _row_spec(ROWB, DOUT)],
        out_shape=[jax.ShapeDtypeStruct((N, DOUT), BF16),
                   jax.ShapeDtypeStruct((N, DOUT), F32)],
    )
    P3, S3 = k2(A2, S2, RCP, Wl3, Wr3, b3.reshape(1, DOUT))

    A3, = _make_agg(N, NCHW, DOUT, False)(P3, src, dst,
                                          jnp.zeros((N, DOUT), BF16))

    k3 = pl.pallas_call(
        _k3_body,
        grid=grid,
        in_specs=[_acc_spec(ROWB, DOUT), _row_spec(ROWB, DOUT), _row_spec(ROWB, 8)],
        out_specs=_row_spec(ROWB, DOUT),
        out_shape=jax.ShapeDtypeStruct((N, DOUT), F32),
    )
    return k3(A3, S3, RCP)[:N0]


# speedup vs baseline: 1.0188x; 1.0042x over previous
import functools

import jax
import jax.numpy as jnp
from jax import lax
from jax.experimental import pallas as pl
from jax.experimental.pallas import tpu as pltpu
from jax.experimental.pallas import tpu_sc as plsc

F32 = jnp.float32
BF16 = jnp.bfloat16
NC, NS = 2, 16
NW = NC * NS
CH = 80


BCH = 16


@functools.lru_cache(maxsize=None)
def _make_count(N: int, NCHW: int):
    RPT = N // NS
    mesh = plsc.VectorSubcoreMesh(core_axis_name="c", subcore_axis_name="s")

    @functools.partial(
        pl.kernel,
        out_type=jax.ShapeDtypeStruct((NC, N, 16), F32),
        mesh=mesh,
        scratch_types=[
            pltpu.VMEM((NCHW, CH), jnp.int32),
            pltpu.VMEM((CH, 16), F32),
            pltpu.VMEM_SHARED((N, 16), F32),
            pltpu.SemaphoreType.DMA,
        ],
        compiler_params=pltpu.CompilerParams(use_tc_tiling_on_sc=False),
    )
    def cnt_kernel(dst_hbm, zero_hbm, ones_hbm, out_hbm, didx, onev, acc, ssem):
        c = lax.axis_index("c")
        s = lax.axis_index("s")
        wid = s * NC + c
        pltpu.sync_copy(zero_hbm.at[pl.ds(s * RPT, RPT)],
                        acc.at[pl.ds(s * RPT, RPT)])
        pltpu.sync_copy(ones_hbm, onev)
        pltpu.sync_copy(dst_hbm.at[wid], didx)
        plsc.subcore_barrier()

        def fire(ch, carry):
            pltpu.async_copy(onev, acc.at[didx.at[ch]], ssem, add=True)
            return carry

        lax.fori_loop(0, NCHW, fire, 0)

        def drain(ch, carry):
            pltpu.make_async_copy(ones_hbm, onev, ssem).wait()
            return carry

        lax.fori_loop(0, NCHW, drain, 0)
        plsc.subcore_barrier()
        pltpu.sync_copy(acc.at[pl.ds(s * RPT, RPT)],
                        out_hbm.at[c, pl.ds(s * RPT, RPT)])

    return cnt_kernel


@functools.lru_cache(maxsize=None)
def _make_agg(N: int, NCHW: int, D: int):
    NBLK = NCHW // BCH
    RPT = N // NS
    NRING = 16
    GLAG = 10
    mesh = plsc.VectorSubcoreMesh(core_axis_name="c", subcore_axis_name="s")

    @functools.partial(
        pl.kernel,
        out_type=[jax.ShapeDtypeStruct((NC, N, D), BF16)],
        mesh=mesh,
        scratch_types=[
            pltpu.VMEM((3, BCH, CH), jnp.int32),
            pltpu.VMEM((3, BCH, CH), jnp.int32),
            pltpu.VMEM((NRING, CH, D), BF16),
            pltpu.VMEM_SHARED((N, D), BF16),
            pltpu.SemaphoreType.DMA,
            pltpu.SemaphoreType.DMA,
        ],
        compiler_params=pltpu.CompilerParams(use_tc_tiling_on_sc=False),
    )
    def agg(p_hbm, src_hbm, dst_hbm, zero_hbm, out_hbm,
            sidx, didx, rows, acc, gsem, ssem):
        c = lax.axis_index("c")
        s = lax.axis_index("s")
        wid = s * NC + c

        def stage_idx(blk, slot):
            pltpu.sync_copy(src_hbm.at[wid, pl.ds(blk * BCH, BCH)],
                            sidx.at[slot])
            pltpu.sync_copy(dst_hbm.at[wid, pl.ds(blk * BCH, BCH)],
                            didx.at[slot])

        pltpu.sync_copy(zero_hbm.at[pl.ds(s * RPT, RPT)],
                        acc.at[pl.ds(s * RPT, RPT)])
        stage_idx(0, 0)
        stage_idx(1, 1)
        plsc.subcore_barrier()

        def idx_of(ch):
            return lax.rem(lax.div(ch, BCH), 3), lax.rem(ch, BCH)

        def gather(ch, b):
            slot, r = idx_of(ch)
            pltpu.async_copy(p_hbm.at[sidx.at[slot, r]], rows.at[b], gsem)

        def scatter(ch, b):
            slot, r = idx_of(ch)
            pltpu.async_copy(rows.at[b], acc.at[didx.at[slot, r]], ssem,
                             add=True)

        def wait_g():
            pltpu.make_async_copy(zero_hbm.at[pl.ds(0, CH)], rows.at[0],
                                  gsem).wait()

        def wait_s():
            pltpu.make_async_copy(zero_hbm.at[pl.ds(0, CH)], rows.at[0],
                                  ssem).wait()

        for k in range(GLAG):
            gather(k, k)

        def body(st, carry):
            blk = lax.div(st, BCH)

            @pl.when(jnp.logical_and(lax.rem(st, BCH) == 0,
                                     blk < NBLK - 1))
            def _():
                stage_idx(blk + 1, lax.rem(blk + 1, 3))

            @pl.when(st >= NRING)
            def _():
                wait_s()

            gather(st, lax.rem(st, NRING))
            wait_g()
            scatter(st - GLAG, lax.rem(st - GLAG, NRING))
            return carry

        lax.fori_loop(GLAG, NCHW, body, 0)
        for j in range(NCHW - GLAG, NCHW):
            wait_g()
            scatter(j, j % NRING)
        for _ in range(NRING):
            wait_s()

        plsc.subcore_barrier()
        pltpu.sync_copy(acc.at[pl.ds(s * RPT, RPT)],
                        out_hbm.at[c, pl.ds(s * RPT, RPT)])

    return agg


def _k0_body(x_ref, wl_ref, wr_ref, b_ref, p_ref, s_ref):
    x = x_ref[...]
    p_ref[...] = jnp.dot(x, wl_ref[...], preferred_element_type=F32).astype(BF16)
    s_ref[...] = jnp.dot(x, wr_ref[...], preferred_element_type=F32) + b_ref[...]


def _k1_body(a_ref, cnt_ref, s1_ref, wl_ref, wr_ref, b_ref,
             p_ref, s_ref, rcp_ref):
    a = a_ref[0].astype(F32) + a_ref[1].astype(F32)
    cnt = cnt_ref[0, :, :1] + cnt_ref[1, :, :1]
    rcp = 1.0 / jnp.maximum(cnt, 1.0)
    h = jnp.maximum(a * rcp + s1_ref[...], 0.0)
    p_ref[...] = jnp.dot(h, wl_ref[...], preferred_element_type=F32).astype(BF16)
    s_ref[...] = jnp.dot(h, wr_ref[...], preferred_element_type=F32) + b_ref[...]
    rcp_ref[...] = jnp.broadcast_to(rcp, (h.shape[0], 8))


def _k2_body(a_ref, s2_ref, rcp_ref, wl_ref, wr_ref, b_ref,
             p_ref, s_ref):
    a = a_ref[0].astype(F32) + a_ref[1].astype(F32)
    h = jnp.maximum(a * rcp_ref[:, :1] + s2_ref[...], 0.0)
    p_ref[...] = jnp.dot(h, wl_ref[...], preferred_element_type=F32).astype(BF16)
    s_ref[...] = jnp.dot(h, wr_ref[...], preferred_element_type=F32) + b_ref[...]


def _k3_body(a_ref, s3_ref, rcp_ref, out_ref):
    z = (a_ref[0].astype(F32) + a_ref[1].astype(F32)) * rcp_ref[:, :1] + s3_ref[...]
    m = jnp.max(z, axis=1, keepdims=True)
    lse = jnp.log(jnp.sum(jnp.exp(z - m), axis=1, keepdims=True)) + m
    out_ref[...] = z - lse


def _row_spec(rb, d):
    return pl.BlockSpec((rb, d), lambda i: (i, 0))


def _acc_spec(rb, d):
    return pl.BlockSpec((NC, rb, d), lambda i: (0, i, 0))


def _full_spec(*shape):
    return pl.BlockSpec(shape, lambda i: tuple(0 for _ in shape))


def kernel(x, edge_index, Wl1, Wr1, b1, Wl2, Wr2, b2, Wl3, Wr3, b3):
    N0, DIN = x.shape
    E = edge_index.shape[1]
    DH = Wl1.shape[1]
    DOUT = Wl3.shape[1]
    N = ((N0 + NS * 8 - 1) // (NS * 8)) * (NS * 8)
    ROWB = N // 8
    grid = (N // ROWB,)
    if N != N0:
        x = jnp.concatenate([x, jnp.zeros((N - N0, DIN), x.dtype)], axis=0)

    EW = E // NW
    NCHW = ((EW + BCH * CH - 1) // (BCH * CH)) * BCH
    src = edge_index[0].reshape(NW, EW)
    dst = edge_index[1].reshape(NW, EW)
    if NCHW * CH != EW:
        npadn = N - N0
        pad_vals = N0 + (jnp.arange(NW * (NCHW * CH - EW), dtype=jnp.int32)
                         % npadn).reshape(NW, NCHW * CH - EW)
        src = jnp.concatenate([src, pad_vals], axis=1)
        dst = jnp.concatenate([dst, pad_vals], axis=1)
    src = src.reshape(NW, NCHW, CH)
    dst = dst.reshape(NW, NCHW, CH)

    k0 = pl.pallas_call(
        _k0_body,
        grid=grid,
        in_specs=[_row_spec(ROWB, DIN), _full_spec(DIN, DH), _full_spec(DIN, DH),
                  _full_spec(1, DH)],
        out_specs=[_row_spec(ROWB, DH), _row_spec(ROWB, DH)],
        out_shape=[jax.ShapeDtypeStruct((N, DH), BF16),
                   jax.ShapeDtypeStruct((N, DH), F32)],
    )
    P1, S1 = k0(x, Wl1, Wr1, b1.reshape(1, DH))

    CNT = _make_count(N, NCHW)(dst, jnp.zeros((N, 16), F32),
                               jnp.ones((CH, 16), F32))
    zeros_f = jnp.zeros((N, DH), BF16)
    agg_f = _make_agg(N, NCHW, DH)
    A1, = agg_f(P1, src, dst, zeros_f)

    k1 = pl.pallas_call(
        _k1_body,
        grid=grid,
        in_specs=[_acc_spec(ROWB, DH), _acc_spec(ROWB, 16),
                  _row_spec(ROWB, DH), _full_spec(DH, DH),
                  _full_spec(DH, DH), _full_spec(1, DH)],
        out_specs=[_row_spec(ROWB, DH), _row_spec(ROWB, DH), _row_spec(ROWB, 8)],
        out_shape=[jax.ShapeDtypeStruct((N, DH), BF16),
                   jax.ShapeDtypeStruct((N, DH), F32),
                   jax.ShapeDtypeStruct((N, 8), F32)],
    )
    P2, S2, RCP = k1(A1, CNT, S1, Wl2, Wr2, b2.reshape(1, DH))

    A2, = agg_f(P2, src, dst, zeros_f)

    k2 = pl.pallas_call(
        _k2_body,
        grid=grid,
        in_specs=[_acc_spec(ROWB, DH),
                  _row_spec(ROWB, DH), _row_spec(ROWB, 8),
                  _full_spec(DH, DOUT), _full_spec(DH, DOUT),
                  _full_spec(1, DOUT)],
        out_specs=[_row_spec(ROWB, DOUT), _row_spec(ROWB, DOUT)],
        out_shape=[jax.ShapeDtypeStruct((N, DOUT), BF16),
                   jax.ShapeDtypeStruct((N, DOUT), F32)],
    )
    P3, S3 = k2(A2, S2, RCP, Wl3, Wr3, b3.reshape(1, DOUT))

    A3, = _make_agg(N, NCHW, DOUT)(P3, src, dst, jnp.zeros((N, DOUT), BF16))

    k3 = pl.pallas_call(
        _k3_body,
        grid=grid,
        in_specs=[_acc_spec(ROWB, DOUT), _row_spec(ROWB, DOUT), _row_spec(ROWB, 8)],
        out_specs=_row_spec(ROWB, DOUT),
        out_shape=jax.ShapeDtypeStruct((N, DOUT), F32),
    )
    return k3(A3, S3, RCP)[:N0]
